# baseline (speedup 1.0000x reference)
import jax
import jax.numpy as jnp
from jax import lax
from jax.experimental import pallas as pl
from jax.experimental.pallas import tpu as pltpu
from jax.experimental.pallas import tpu_sc as plsc

_M = 1000000
_D = 64
_B = 16384
_NS = 16
_NC = 2
_CHUNK = _B // _NS
_ROWS = 8
_RL = _CHUNK // _ROWS
_NV = _RL // 16
_BITS = 14
_GARBAGE = _M


def _sc_body(val_hbm, idx_hbm, nl_hbm, out_img, out_lbl,
             idx2d, ival2d, abuf, cbuf, tbuf, zbuf, lblbuf, rowbuf, T, sem):
    cid = lax.axis_index("c")
    sid = lax.axis_index("s")
    base = sid * _CHUNK

    for r in range(_ROWS):
        pltpu.sync_copy(idx_hbm.at[pl.ds(base + r * _RL, _RL)], idx2d.at[r])

    lane = lax.iota(jnp.int32, 16)
    one = jnp.full((16,), 1, jnp.int32)
    zero = jnp.full((16,), 0, jnp.int32)
    for r in range(_ROWS):
        for v in range(_NV):
            sl = pl.ds(v * 16, 16)
            ival2d[r, sl] = lane + (base + r * _RL + v * 16)
            abuf[r, sl] = one
            zbuf[r, sl] = zero

    def round_body(t, carry):
        b = (_BITS - 1) - t
        cps = [pltpu.async_copy(zbuf.at[r], T.at[idx2d.at[r]], sem)
               for r in range(_ROWS)]
        for c in cps:
            c.wait()
        plsc.subcore_barrier()
        for r in range(_ROWS):
            for v in range(_NV):
                sl = pl.ds(v * 16, 16)
                bit = lax.shift_right_logical(ival2d[r, sl],
                                              jnp.broadcast_to(b, (16,))) & one
                cbuf[r, sl] = abuf[r, sl] * bit
        cps = [pltpu.async_copy(cbuf.at[r], T.at[idx2d.at[r]], sem, add=True)
               for r in range(_ROWS)]
        for c in cps:
            c.wait()
        plsc.subcore_barrier()
        cps = [pltpu.async_copy(T.at[idx2d.at[r]], tbuf.at[r], sem)
               for r in range(_ROWS)]
        for c in cps:
            c.wait()
        for r in range(_ROWS):
            for v in range(_NV):
                sl = pl.ds(v * 16, 16)
                bit = lax.shift_right_logical(ival2d[r, sl],
                                              jnp.broadcast_to(b, (16,))) & one
                tpos = jnp.minimum(tbuf[r, sl], one)
                keep = jnp.maximum(bit, one - tpos)
                abuf[r, sl] = abuf[r, sl] * keep
        plsc.subcore_barrier()
        return carry

    lax.fori_loop(0, _BITS, round_body, jnp.int32(0))

    cps = [pltpu.async_copy(zbuf.at[r], T.at[idx2d.at[r]], sem)
           for r in range(_ROWS)]
    for c in cps:
        c.wait()
    plsc.subcore_barrier()
    for r in range(_ROWS):
        for v in range(_NV):
            sl = pl.ds(v * 16, 16)
            cbuf[r, sl] = abuf[r, sl] * ival2d[r, sl]
    cps = [pltpu.async_copy(cbuf.at[r], T.at[idx2d.at[r]], sem, add=True)
           for r in range(_ROWS)]
    for c in cps:
        c.wait()
    plsc.subcore_barrier()
    cps = [pltpu.async_copy(T.at[idx2d.at[r]], tbuf.at[r], sem)
           for r in range(_ROWS)]
    for c in cps:
        c.wait()

    del cid
    for r in range(_ROWS):
        off = base + r * _RL
        pltpu.async_copy(nl_hbm.at[tbuf.at[r]], lblbuf, sem).wait()
        pltpu.sync_copy(lblbuf, out_lbl.at[pl.ds(off, _RL)])
        pltpu.async_copy(val_hbm.at[tbuf.at[r]], rowbuf, sem).wait()
        pltpu.sync_copy(rowbuf, out_img.at[pl.ds(off, _RL), :])


def kernel(mem, val, mem_labels, idx, new_labels):
    del mem, mem_labels
    f = pl.kernel(
        _sc_body,
        out_type=(jax.ShapeDtypeStruct((_B, 128), jnp.float32),
                  jax.ShapeDtypeStruct((_B,), jnp.int32)),
        mesh=plsc.VectorSubcoreMesh(core_axis_name="c", subcore_axis_name="s",
                                    num_cores=1),
        scratch_types=[
            pltpu.VMEM((_ROWS, _RL), jnp.int32),
            pltpu.VMEM((_ROWS, _RL), jnp.int32),
            pltpu.VMEM((_ROWS, _RL), jnp.int32),
            pltpu.VMEM((_ROWS, _RL), jnp.int32),
            pltpu.VMEM((_ROWS, _RL), jnp.int32),
            pltpu.VMEM((_ROWS, _RL), jnp.int32),
            pltpu.VMEM((_RL,), jnp.int32),
            pltpu.VMEM((_RL, 128), jnp.float32),
            pltpu.VMEM_SHARED((_M + 16,), jnp.int32),
            pltpu.SemaphoreType.DMA,
        ],
    )
    val_p = jnp.pad(val, ((0, 0), (0, 128 - _D)))
    ret_imgs_p, ret_labels = f(val_p, idx, new_labels)
    return (ret_imgs_p[:, :_D], ret_labels)


# speedup vs baseline: 1.4452x; 1.4452x over previous
import jax
import jax.numpy as jnp
from jax import lax
from jax.experimental import pallas as pl
from jax.experimental.pallas import tpu as pltpu
from jax.experimental.pallas import tpu_sc as plsc

_M = 1000000
_D = 64
_B = 16384
_NS = 16
_NC = 2
_CHUNK = _B // _NS
_ROWS = 8
_RL = _CHUNK // _ROWS
_NV = _RL // 16
_BITS = 14


def _sc_body(val_hbm, idx_hbm, nl_hbm, out_img, out_lbl,
             idx2d, ival2d, obuf, zbuf, cnt2d, sum2d, wbuf, abuf, cbuf, tbuf,
             fwr, frd, lblbuf, rowbuf, T0, F, sem):
    cid = lax.axis_index("c")
    sid = lax.axis_index("s")
    base = sid * _CHUNK

    for r in range(_ROWS):
        pltpu.sync_copy(idx_hbm.at[pl.ds(base + r * _RL, _RL)], idx2d.at[r])

    lane = lax.iota(jnp.int32, 16)
    one = jnp.full((16,), 1, jnp.int32)
    zero = jnp.full((16,), 0, jnp.int32)
    for r in range(_ROWS):
        for v in range(_NV):
            sl = pl.ds(v * 16, 16)
            ival2d[r, sl] = lane + (base + r * _RL + v * 16)
            obuf[r, sl] = one
            zbuf[r, sl] = zero

    for src, dst in ((obuf, cnt2d), (ival2d, sum2d)):
        cps = [pltpu.async_copy(zbuf.at[r], T0.at[idx2d.at[r]], sem)
               for r in range(_ROWS)]
        for c in cps:
            c.wait()
        plsc.subcore_barrier()
        cps = [pltpu.async_copy(src.at[r], T0.at[idx2d.at[r]], sem, add=True)
               for r in range(_ROWS)]
        for c in cps:
            c.wait()
        plsc.subcore_barrier()
        cps = [pltpu.async_copy(T0.at[idx2d.at[r]], dst.at[r], sem)
               for r in range(_ROWS)]
        for c in cps:
            c.wait()
        plsc.subcore_barrier()

    acc = jnp.zeros((16,), jnp.int32)
    for r in range(_ROWS):
        for v in range(_NV):
            sl = pl.ds(v * 16, 16)
            c_ = cnt2d[r, sl]
            iv = ival2d[r, sl]
            pair = jnp.maximum(iv, sum2d[r, sl] - iv)
            wbuf[r, sl] = jnp.where(c_ == 2, pair, iv)
            big = jnp.minimum(jnp.maximum(c_ - 2, 0), 1)
            abuf[r, sl] = big
            acc = jnp.maximum(acc, big)
    fwr[...] = acc
    pltpu.async_copy(fwr, F.at[pl.ds(sid * 16, 16)], sem).wait()
    plsc.subcore_barrier()
    pltpu.async_copy(F, frd, sem).wait()
    acc2 = jnp.zeros((16,), jnp.int32)
    for v in range(_NS):
        acc2 = jnp.maximum(acc2, frd[pl.ds(v * 16, 16)])
    for sh in (8, 4, 2, 1):
        acc2 = jnp.maximum(acc2, jnp.take(acc2, lane ^ sh))
    need_tournament = acc2[0]

    @pl.when(need_tournament > 0)
    def _tournament():
        def round_body(t, carry):
            b = (_BITS - 1) - t
            cps = [pltpu.async_copy(zbuf.at[r], T0.at[idx2d.at[r]], sem)
                   for r in range(_ROWS)]
            for c in cps:
                c.wait()
            plsc.subcore_barrier()
            for r in range(_ROWS):
                for v in range(_NV):
                    sl = pl.ds(v * 16, 16)
                    bit = lax.shift_right_logical(
                        ival2d[r, sl], jnp.broadcast_to(b, (16,))) & one
                    cbuf[r, sl] = abuf[r, sl] * bit
            cps = [pltpu.async_copy(cbuf.at[r], T0.at[idx2d.at[r]], sem,
                                    add=True) for r in range(_ROWS)]
            for c in cps:
                c.wait()
            plsc.subcore_barrier()
            cps = [pltpu.async_copy(T0.at[idx2d.at[r]], tbuf.at[r], sem)
                   for r in range(_ROWS)]
            for c in cps:
                c.wait()
            for r in range(_ROWS):
                for v in range(_NV):
                    sl = pl.ds(v * 16, 16)
                    bit = lax.shift_right_logical(
                        ival2d[r, sl], jnp.broadcast_to(b, (16,))) & one
                    tpos = jnp.minimum(tbuf[r, sl], one)
                    keep = jnp.maximum(bit, one - tpos)
                    abuf[r, sl] = abuf[r, sl] * keep
            plsc.subcore_barrier()
            return carry

        lax.fori_loop(0, _BITS, round_body, jnp.int32(0))

        cps = [pltpu.async_copy(zbuf.at[r], T0.at[idx2d.at[r]], sem)
               for r in range(_ROWS)]
        for c in cps:
            c.wait()
        plsc.subcore_barrier()
        for r in range(_ROWS):
            for v in range(_NV):
                sl = pl.ds(v * 16, 16)
                cbuf[r, sl] = abuf[r, sl] * ival2d[r, sl]
        cps = [pltpu.async_copy(cbuf.at[r], T0.at[idx2d.at[r]], sem, add=True)
               for r in range(_ROWS)]
        for c in cps:
            c.wait()
        plsc.subcore_barrier()
        cps = [pltpu.async_copy(T0.at[idx2d.at[r]], tbuf.at[r], sem)
               for r in range(_ROWS)]
        for c in cps:
            c.wait()
        for r in range(_ROWS):
            for v in range(_NV):
                sl = pl.ds(v * 16, 16)
                wbuf[r, sl] = jnp.where(cnt2d[r, sl] >= 3,
                                        tbuf[r, sl], wbuf[r, sl])

    for h in range(_ROWS // _NC):
        r = cid * (_ROWS // _NC) + h
        off = base + r * _RL
        pltpu.async_copy(nl_hbm.at[wbuf.at[r]], lblbuf, sem).wait()
        pltpu.sync_copy(lblbuf, out_lbl.at[pl.ds(off, _RL)])
        pltpu.async_copy(val_hbm.at[wbuf.at[r]], rowbuf, sem).wait()
        pltpu.sync_copy(rowbuf, out_img.at[pl.ds(off, _RL), :])


def kernel(mem, val, mem_labels, idx, new_labels):
    del mem, mem_labels
    f = pl.kernel(
        _sc_body,
        out_type=(jax.ShapeDtypeStruct((_B, 128), jnp.float32),
                  jax.ShapeDtypeStruct((_B,), jnp.int32)),
        mesh=plsc.VectorSubcoreMesh(core_axis_name="c", subcore_axis_name="s"),
        scratch_types=[
            pltpu.VMEM((_ROWS, _RL), jnp.int32),
            pltpu.VMEM((_ROWS, _RL), jnp.int32),
            pltpu.VMEM((_ROWS, _RL), jnp.int32),
            pltpu.VMEM((_ROWS, _RL), jnp.int32),
            pltpu.VMEM((_ROWS, _RL), jnp.int32),
            pltpu.VMEM((_ROWS, _RL), jnp.int32),
            pltpu.VMEM((_ROWS, _RL), jnp.int32),
            pltpu.VMEM((_ROWS, _RL), jnp.int32),
            pltpu.VMEM((_ROWS, _RL), jnp.int32),
            pltpu.VMEM((_ROWS, _RL), jnp.int32),
            pltpu.VMEM((16,), jnp.int32),
            pltpu.VMEM((16 * _NS,), jnp.int32),
            pltpu.VMEM((_RL,), jnp.int32),
            pltpu.VMEM((_RL, 128), jnp.float32),
            pltpu.VMEM_SHARED((_M + 16,), jnp.int32),
            pltpu.VMEM_SHARED((16 * _NS,), jnp.int32),
            pltpu.SemaphoreType.DMA,
        ],
    )
    val_p = jnp.pad(val, ((0, 0), (0, 128 - _D)))
    ret_imgs_p, ret_labels = f(val_p, idx, new_labels)
    return (ret_imgs_p[:, :_D], ret_labels)


# speedup vs baseline: 1.7315x; 1.1982x over previous
import jax
import jax.numpy as jnp
from jax import lax
from jax.experimental import pallas as pl
from jax.experimental.pallas import tpu as pltpu
from jax.experimental.pallas import tpu_sc as plsc

_M = 1000000
_D = 64
_B = 16384
_NS = 16
_NC = 2
_CHUNK = _B // _NS
_ROWS = 8
_RL = _CHUNK // _ROWS
_NV = _RL // 16
_BITS = 14


def _sc_body(val_hbm, idx_hbm, nl_hbm, out_img, out_lbl,
             idx2d, ival2d, obuf, zbuf, cnt2d, sum2d, wbuf, abuf, cbuf, tbuf,
             fwr, frd, lblbuf, rowbuf, T0, F, sem, sem2, sem3):
    cid = lax.axis_index("c")
    sid = lax.axis_index("s")
    base = sid * _CHUNK

    cps = [pltpu.async_copy(idx_hbm.at[pl.ds(base + r * _RL, _RL)],
                            idx2d.at[r], sem) for r in range(_ROWS)]

    lane = lax.iota(jnp.int32, 16)
    one = jnp.full((16,), 1, jnp.int32)
    zero = jnp.full((16,), 0, jnp.int32)
    for r in range(_ROWS):
        for v in range(_NV):
            sl = pl.ds(v * 16, 16)
            ival2d[r, sl] = lane + (base + r * _RL + v * 16)
            obuf[r, sl] = one
            zbuf[r, sl] = zero
    for c in cps:
        c.wait()

    cps = [pltpu.async_copy(zbuf.at[r], T0.at[idx2d.at[r]], sem)
           for r in range(_ROWS)]
    for c in cps:
        c.wait()
    plsc.subcore_barrier()
    cps = [pltpu.async_copy(obuf.at[r], T0.at[idx2d.at[r]], sem, add=True)
           for r in range(_ROWS)]
    for c in cps:
        c.wait()
    plsc.subcore_barrier()
    cps = [pltpu.async_copy(T0.at[idx2d.at[r]], cnt2d.at[r], sem)
           for r in range(_ROWS)]
    for c in cps:
        c.wait()
    acc = jnp.zeros((16,), jnp.int32)
    for r in range(_ROWS):
        for v in range(_NV):
            sl = pl.ds(v * 16, 16)
            big = jnp.minimum(jnp.maximum(cnt2d[r, sl] - 2, 0), 1)
            abuf[r, sl] = big
            acc = jnp.maximum(acc, big)
    fwr[...] = acc
    plsc.subcore_barrier()
    cps = ([pltpu.async_copy(ival2d.at[r], T0.at[idx2d.at[r]], sem, add=True)
            for r in range(_ROWS)] +
           [pltpu.async_copy(fwr, F.at[pl.ds(sid * 16, 16)], sem)])
    for c in cps:
        c.wait()
    plsc.subcore_barrier()
    cps = ([pltpu.async_copy(T0.at[idx2d.at[r]], sum2d.at[r], sem)
            for r in range(_ROWS)] +
           [pltpu.async_copy(F, frd, sem)])
    for c in cps:
        c.wait()
    for r in range(_ROWS):
        for v in range(_NV):
            sl = pl.ds(v * 16, 16)
            c_ = cnt2d[r, sl]
            iv = ival2d[r, sl]
            pair = jnp.maximum(iv, sum2d[r, sl] - c_ - iv)
            wbuf[r, sl] = jnp.where(c_ == 2, pair, iv)
    acc2 = jnp.zeros((16,), jnp.int32)
    for v in range(_NS):
        acc2 = jnp.maximum(acc2, frd[pl.ds(v * 16, 16)])
    for sh in (8, 4, 2, 1):
        acc2 = jnp.maximum(acc2, jnp.take(acc2, lane ^ sh))
    need_tournament = acc2[0]

    @pl.when(need_tournament > 0)
    def _tournament():
        plsc.subcore_barrier()

        def round_body(t, carry):
            b = (_BITS - 1) - t
            cps = [pltpu.async_copy(zbuf.at[r], T0.at[idx2d.at[r]], sem)
                   for r in range(_ROWS)]
            for c in cps:
                c.wait()
            plsc.subcore_barrier()
            for r in range(_ROWS):
                for v in range(_NV):
                    sl = pl.ds(v * 16, 16)
                    bit = lax.shift_right_logical(
                        ival2d[r, sl], jnp.broadcast_to(b, (16,))) & one
                    cbuf[r, sl] = abuf[r, sl] * bit
            cps = [pltpu.async_copy(cbuf.at[r], T0.at[idx2d.at[r]], sem,
                                    add=True) for r in range(_ROWS)]
            for c in cps:
                c.wait()
            plsc.subcore_barrier()
            cps = [pltpu.async_copy(T0.at[idx2d.at[r]], tbuf.at[r], sem)
                   for r in range(_ROWS)]
            for c in cps:
                c.wait()
            for r in range(_ROWS):
                for v in range(_NV):
                    sl = pl.ds(v * 16, 16)
                    bit = lax.shift_right_logical(
                        ival2d[r, sl], jnp.broadcast_to(b, (16,))) & one
                    tpos = jnp.minimum(tbuf[r, sl], one)
                    keep = jnp.maximum(bit, one - tpos)
                    abuf[r, sl] = abuf[r, sl] * keep
            plsc.subcore_barrier()
            return carry

        lax.fori_loop(0, _BITS, round_body, jnp.int32(0))

        cps = [pltpu.async_copy(zbuf.at[r], T0.at[idx2d.at[r]], sem)
               for r in range(_ROWS)]
        for c in cps:
            c.wait()
        plsc.subcore_barrier()
        for r in range(_ROWS):
            for v in range(_NV):
                sl = pl.ds(v * 16, 16)
                cbuf[r, sl] = abuf[r, sl] * ival2d[r, sl]
        cps = [pltpu.async_copy(cbuf.at[r], T0.at[idx2d.at[r]], sem, add=True)
               for r in range(_ROWS)]
        for c in cps:
            c.wait()
        plsc.subcore_barrier()
        cps = [pltpu.async_copy(T0.at[idx2d.at[r]], tbuf.at[r], sem)
               for r in range(_ROWS)]
        for c in cps:
            c.wait()
        for r in range(_ROWS):
            for v in range(_NV):
                sl = pl.ds(v * 16, 16)
                wbuf[r, sl] = jnp.where(cnt2d[r, sl] >= 3,
                                        tbuf[r, sl], wbuf[r, sl])

    nh = _ROWS // _NC
    half = cid * nh
    lcps = [pltpu.async_copy(nl_hbm.at[wbuf.at[half + h]], lblbuf.at[h], sem)
            for h in range(nh)]
    rcps = [None] * nh
    rcps[0] = pltpu.async_copy(val_hbm.at[wbuf.at[half]], rowbuf.at[0], sem2)
    for c in lcps:
        c.wait()
    for h in range(nh):
        pltpu.sync_copy(lblbuf.at[h],
                        out_lbl.at[pl.ds(base + (half + h) * _RL, _RL)])
    for h in range(nh):
        if h + 1 < nh:
            rcps[h + 1] = pltpu.async_copy(val_hbm.at[wbuf.at[half + h + 1]],
                                           rowbuf.at[(h + 1) % 2],
                                           sem2 if (h + 1) % 2 == 0 else sem3)
        rcps[h].wait()
        pltpu.sync_copy(rowbuf.at[h % 2],
                        out_img.at[pl.ds(base + (half + h) * _RL, _RL), :])


def kernel(mem, val, mem_labels, idx, new_labels):
    del mem, mem_labels
    f = pl.kernel(
        _sc_body,
        out_type=(jax.ShapeDtypeStruct((_B, 128), jnp.float32),
                  jax.ShapeDtypeStruct((_B,), jnp.int32)),
        mesh=plsc.VectorSubcoreMesh(core_axis_name="c", subcore_axis_name="s"),
        scratch_types=[
            pltpu.VMEM((_ROWS, _RL), jnp.int32),
            pltpu.VMEM((_ROWS, _RL), jnp.int32),
            pltpu.VMEM((_ROWS, _RL), jnp.int32),
            pltpu.VMEM((_ROWS, _RL), jnp.int32),
            pltpu.VMEM((_ROWS, _RL), jnp.int32),
            pltpu.VMEM((_ROWS, _RL), jnp.int32),
            pltpu.VMEM((_ROWS, _RL), jnp.int32),
            pltpu.VMEM((_ROWS, _RL), jnp.int32),
            pltpu.VMEM((_ROWS, _RL), jnp.int32),
            pltpu.VMEM((_ROWS, _RL), jnp.int32),
            pltpu.VMEM((16,), jnp.int32),
            pltpu.VMEM((16 * _NS,), jnp.int32),
            pltpu.VMEM((_ROWS // _NC, _RL), jnp.int32),
            pltpu.VMEM((2, _RL, 128), jnp.float32),
            pltpu.VMEM_SHARED((_M + 16,), jnp.int32),
            pltpu.VMEM_SHARED((16 * _NS,), jnp.int32),
            pltpu.SemaphoreType.DMA,
            pltpu.SemaphoreType.DMA,
            pltpu.SemaphoreType.DMA,
        ],
    )
    val_p = jnp.pad(val, ((0, 0), (0, 128 - _D)))
    ret_imgs_p, ret_labels = f(val_p, idx, new_labels)
    return (ret_imgs_p[:, :_D], ret_labels)
